# baseline (speedup 1.0000x reference)
import functools

import jax
import jax.numpy as jnp
from jax import lax
from jax.experimental import pallas as pl
from jax.experimental.pallas import tpu as pltpu
from jax.experimental.pallas import tpu_sc as plsc

_INTERPRET = False

B, S, H, NH, E, TOPK, FFN = 1, 2048, 1024, 16, 8, 2, 2048
DH = H // NH
T = B * S
A = T * TOPK
BM = 128
APAD = A + E * BM
NCHUNK = APAD // BM
LANES = 128

SC_CORES, SC_SUBCORES = 2, 16
SC_TILES = SC_CORES * SC_SUBCORES

BQ = 512
BS = 256


_BN = 512


def _rope(y, c, s):
    lane = lax.broadcasted_iota(jnp.int32, y.shape, 1)
    even = (lane & 1) == 0
    r_minus = pltpu.roll(y, _BN - 1, 1)
    r_plus = pltpu.roll(y, 1, 1)
    return y * c + jnp.where(even, -r_minus, r_plus) * s


def _qkv_body(x_ref, g_ref, wq_ref, wk_ref, wv_ref,
              cos_ref, sin_ref, q_ref, k_ref, v_ref):
    xv = x_ref[...]
    ms = jnp.mean(xv * xv, axis=-1, keepdims=True)
    h = xv * lax.rsqrt(ms + 1e-6) * g_ref[...]
    c = cos_ref[...]
    s = sin_ref[...]
    q_ref[...] = _rope(jnp.dot(h, wq_ref[...],
                               preferred_element_type=jnp.float32), c, s)
    k_ref[...] = _rope(jnp.dot(h, wk_ref[...],
                               preferred_element_type=jnp.float32), c, s)
    v_ref[...] = jnp.dot(h, wv_ref[...], preferred_element_type=jnp.float32)


def _qkv(x2d, g, wq, wk, wv, cos_t, sin_t):
    grid = (S // BS, H // _BN)
    row = pl.BlockSpec((BS, H), lambda i, j: (i, 0))
    wcol = pl.BlockSpec((H, _BN), lambda i, j: (0, j))
    tile = pl.BlockSpec((BS, _BN), lambda i, j: (i, j))
    gspec = pl.BlockSpec((1, H), lambda i, j: (0, 0))
    return pl.pallas_call(
        _qkv_body,
        grid=grid,
        in_specs=[row, gspec, wcol, wcol, wcol, tile, tile],
        out_specs=[tile, tile, tile],
        out_shape=[jax.ShapeDtypeStruct((S, H), jnp.float32)] * 3,
        interpret=_INTERPRET,
    )(x2d, g, wq, wk, wv, cos_t, sin_t)


def _flash_step(masked, q_ref, k_ref, v_ref, m_ref, l_ref, acc_ref):
    s = lax.dot_general(q_ref[0], k_ref[0], (((1,), (1,)), ((), ())),
                        preferred_element_type=jnp.float32)
    s = s * (1.0 / (DH ** 0.5))
    if masked:
        s = jnp.where(lax.broadcasted_iota(jnp.int32, (BQ, BQ), 1)
                      <= lax.broadcasted_iota(jnp.int32, (BQ, BQ), 0),
                      s, -1e9)
    m_old = m_ref[:, 0:1]
    m_new = jnp.maximum(m_old, jnp.max(s, axis=-1, keepdims=True))
    p = jnp.exp(s - m_new)
    corr = jnp.exp(m_old - m_new)
    l_new = l_ref[:, 0:1] * corr + jnp.sum(p, axis=-1, keepdims=True)
    m_ref[...] = jnp.broadcast_to(m_new, m_ref.shape)
    l_ref[...] = jnp.broadcast_to(l_new, l_ref.shape)
    acc_ref[...] = acc_ref[...] * corr + jnp.dot(
        p, v_ref[0], preferred_element_type=jnp.float32)


def _attn_body(q_ref, k_ref, v_ref, o_ref, m_ref, l_ref, acc_ref):
    qb = pl.program_id(1)
    kb = pl.program_id(2)

    @pl.when(kb == 0)
    def _():
        m_ref[...] = jnp.full_like(m_ref, -1e30)
        l_ref[...] = jnp.zeros_like(l_ref)
        acc_ref[...] = jnp.zeros_like(acc_ref)

    @pl.when(kb < qb)
    def _():
        _flash_step(False, q_ref, k_ref, v_ref, m_ref, l_ref, acc_ref)

    @pl.when(kb == qb)
    def _():
        _flash_step(True, q_ref, k_ref, v_ref, m_ref, l_ref, acc_ref)
        o_ref[0] = acc_ref[...] / l_ref[:, 0:1]


def _attention(qh, kh, vh):
    NB = S // BQ
    grid = (NH, NB, NB)
    qspec = pl.BlockSpec((1, BQ, DH), lambda h, i, j: (h, i, 0))
    kvspec = pl.BlockSpec((1, BQ, DH), lambda h, i, j: (h, jnp.minimum(i, j), 0))
    return pl.pallas_call(
        _attn_body,
        grid=grid,
        in_specs=[qspec, kvspec, kvspec],
        out_specs=qspec,
        out_shape=jax.ShapeDtypeStruct((NH, S, DH), jnp.float32),
        scratch_shapes=[pltpu.VMEM((BQ, 128), jnp.float32),
                        pltpu.VMEM((BQ, 128), jnp.float32),
                        pltpu.VMEM((BQ, DH), jnp.float32)],
        interpret=_INTERPRET,
    )(qh, kh, vh)


def _router_body(x_ref, o_ref, wo_ref, g2_ref, wr_ref,
                 xo_ref, h2_ref, meta_ref, gate_ref, psum_ref, cnt_ref):
    i = pl.program_id(0)
    xo = x_ref[...] + jnp.dot(o_ref[...], wo_ref[...],
                              preferred_element_type=jnp.float32)
    xo_ref[...] = xo
    ms = jnp.mean(xo * xo, axis=-1, keepdims=True)
    h2 = xo * lax.rsqrt(ms + 1e-6) * g2_ref[...]
    h2_ref[...] = h2
    lg = jnp.dot(h2, wr_ref[...], preferred_element_type=jnp.float32)
    lane = lax.broadcasted_iota(jnp.int32, (BS, LANES), 1)
    lg = jnp.where(lane < E, lg, -1e30)
    mx = jnp.max(lg, axis=-1, keepdims=True)
    p = jnp.exp(lg - mx)
    p = p / jnp.sum(p, axis=-1, keepdims=True)
    m1 = jnp.max(p, axis=-1, keepdims=True)
    i1 = jnp.min(jnp.where(p == m1, lane, LANES), axis=-1, keepdims=True)
    pm = jnp.where(lane == i1, -1.0, p)
    m2 = jnp.max(pm, axis=-1, keepdims=True)
    i2 = jnp.min(jnp.where(pm == m2, lane, LANES), axis=-1, keepdims=True)
    den = m1 + m2
    g0 = m1 / den
    g1 = m2 / den
    onehot = (lane == i1).astype(jnp.float32) + (lane == i2).astype(jnp.float32)
    rr = lax.broadcasted_iota(jnp.int32, (BS, BS), 0)
    cc = lax.broadcasted_iota(jnp.int32, (BS, BS), 1)
    tril = (cc <= rr).astype(jnp.float32)
    cum = jnp.dot(tril, onehot, preferred_element_type=jnp.float32)

    @pl.when(i == 0)
    def _():
        psum_ref[...] = jnp.zeros_like(psum_ref)
        cnt_ref[...] = jnp.zeros_like(cnt_ref)

    base = cnt_ref[...]
    tot = cum + base
    rank0 = jnp.sum(jnp.where(lane == i1, tot, 0.0), axis=-1, keepdims=True) - 1.0
    rank1 = jnp.sum(jnp.where(lane == i2, tot, 0.0), axis=-1, keepdims=True) - 1.0
    cnt_ref[...] = base + cum[BS - 1:BS, :]
    psum_ref[...] = psum_ref[...] + jnp.sum(p, axis=0, keepdims=True)
    meta = jnp.where(lane == 0, i1,
           jnp.where(lane == 1, i2,
           jnp.where(lane == 2, rank0.astype(jnp.int32),
           jnp.where(lane == 3, rank1.astype(jnp.int32), 0))))
    meta_ref[...] = meta
    gate_ref[...] = jnp.where(lane == 0, g0,
                    jnp.where(lane == 1, g1, 0.0))


def _router(x2d, attn_o, wo, g2, wr_pad):
    grid = (S // BS,)
    row = pl.BlockSpec((BS, H), lambda i: (i, 0))
    meta = pl.BlockSpec((BS, LANES), lambda i: (i, 0))
    acc = pl.BlockSpec((1, LANES), lambda i: (0, 0))
    return pl.pallas_call(
        _router_body,
        grid=grid,
        in_specs=[row, row,
                  pl.BlockSpec((H, H), lambda i: (0, 0)),
                  pl.BlockSpec((1, H), lambda i: (0, 0)),
                  pl.BlockSpec((H, LANES), lambda i: (0, 0))],
        out_specs=[row, row, meta, meta, acc, acc],
        out_shape=[jax.ShapeDtypeStruct((S, H), jnp.float32),
                   jax.ShapeDtypeStruct((S, H), jnp.float32),
                   jax.ShapeDtypeStruct((S, LANES), jnp.int32),
                   jax.ShapeDtypeStruct((S, LANES), jnp.float32),
                   jax.ShapeDtypeStruct((1, LANES), jnp.float32),
                   jax.ShapeDtypeStruct((1, LANES), jnp.float32)],
        interpret=_INTERPRET,
    )(x2d, attn_o, wo, g2, wr_pad)


def _ffn_body(e_ref, x_ref, w1_ref, w2_ref, o_ref):
    hmid = jnp.dot(x_ref[...], w1_ref[0], preferred_element_type=jnp.float32)
    hmid = hmid * (1.0 / (1.0 + jnp.exp(-hmid)))
    o_ref[...] = jnp.dot(hmid, w2_ref[0], preferred_element_type=jnp.float32)


def _grouped_ffn(h2s, w1, w2, e_idx):
    grid_spec = pltpu.PrefetchScalarGridSpec(
        num_scalar_prefetch=1,
        grid=(NCHUNK,),
        in_specs=[
            pl.BlockSpec((BM, H), lambda i, e_ref: (i, 0)),
            pl.BlockSpec((1, H, FFN), lambda i, e_ref: (e_ref[i], 0, 0)),
            pl.BlockSpec((1, FFN, H), lambda i, e_ref: (e_ref[i], 0, 0)),
        ],
        out_specs=pl.BlockSpec((BM, H), lambda i, e_ref: (i, 0)),
    )
    return pl.pallas_call(
        _ffn_body,
        grid_spec=grid_spec,
        out_shape=jax.ShapeDtypeStruct((APAD, H), jnp.float32),
        interpret=_INTERPRET,
    )(e_idx, h2s, w1, w2)


def _combine_body(xo_ref, rk_ref, gate_ref, out_ref):
    g0 = gate_ref[:, 0:1]
    g1 = gate_ref[:, 1:2]
    out_ref[...] = xo_ref[...] + g0 * rk_ref[:, 0, :] + g1 * rk_ref[:, 1, :]


def _combine(xo, rk3, gates):
    grid = (S // BS,)
    row = pl.BlockSpec((BS, H), lambda i: (i, 0))
    return pl.pallas_call(
        _combine_body,
        grid=grid,
        in_specs=[row,
                  pl.BlockSpec((BS, TOPK, H), lambda i: (i, 0, 0)),
                  pl.BlockSpec((BS, LANES), lambda i: (i, 0))],
        out_specs=row,
        out_shape=jax.ShapeDtypeStruct((S, H), jnp.float32),
        interpret=_INTERPRET,
    )(xo, rk3, gates)


_PER_TILE = APAD // SC_TILES
_GCHUNK = _PER_TILE // 2
_PER_SUB = A // SC_SUBCORES
_ZSLICE = APAD // SC_SUBCORES


def _dispatch_body(topi_hbm, rank_hbm, off_hbm, h2_hbm, h2s_hbm, dest_hbm,
                   topi_v, rank_v, off_v, dest_v, tok_v, zero_v,
                   shared_sort, myidx_v, rows_v, sem):
    cid = lax.axis_index("c")
    sid = lax.axis_index("s")
    wid = sid * SC_CORES + cid

    abase = sid * _PER_SUB
    pltpu.sync_copy(topi_hbm.at[pl.ds(abase, _PER_SUB)], topi_v)
    pltpu.sync_copy(rank_hbm.at[pl.ds(abase, _PER_SUB)], rank_v)
    pltpu.sync_copy(off_hbm, off_v)

    def zbody(i, carry):
        zero_v[pl.ds(i * 16, 16)] = jnp.zeros((16,), jnp.int32)
        return carry
    lax.fori_loop(0, _ZSLICE // 16, zbody, 0)
    pltpu.sync_copy(zero_v, shared_sort.at[pl.ds(sid * _ZSLICE, _ZSLICE)])

    def body(i, carry):
        e = topi_v[pl.ds(i * 16, 16)]
        r = rank_v[pl.ds(i * 16, 16)]
        d = plsc.load_gather(off_v, [e]) + r
        dest_v[pl.ds(i * 16, 16)] = d
        a = abase + i * 16 + lax.iota(jnp.int32, 16)
        tok_v[pl.ds(i * 16, 16)] = a >> 1
        return carry
    lax.fori_loop(0, _PER_SUB // 16, body, 0)

    @pl.when(cid == 0)
    def _():
        pltpu.sync_copy(dest_v, dest_hbm.at[pl.ds(abase, _PER_SUB)])

    plsc.subcore_barrier()
    pltpu.sync_copy(tok_v, shared_sort.at[dest_v], add=True)
    plsc.subcore_barrier()

    base = wid * _PER_TILE
    pltpu.sync_copy(shared_sort.at[pl.ds(base, _PER_TILE)], myidx_v)
    for ch in range(_PER_TILE // _GCHUNK):
        pltpu.async_copy(h2_hbm.at[myidx_v.at[pl.ds(ch * _GCHUNK, _GCHUNK)]],
                         rows_v, sem).wait()
        pltpu.sync_copy(rows_v, h2s_hbm.at[pl.ds(base + ch * _GCHUNK, _GCHUNK)])


def _dispatch(topi_flat, rank_flat, off16, h2):
    mesh = plsc.VectorSubcoreMesh(core_axis_name="c", subcore_axis_name="s")
    fn = functools.partial(
        pl.kernel, _dispatch_body, mesh=mesh,
        compiler_params=pltpu.CompilerParams(needs_layout_passes=False),
        out_type=[jax.ShapeDtypeStruct((APAD, H), jnp.float32),
                  jax.ShapeDtypeStruct((A,), jnp.int32)],
        scratch_types=[
            pltpu.VMEM((_PER_SUB,), jnp.int32),
            pltpu.VMEM((_PER_SUB,), jnp.int32),
            pltpu.VMEM((16,), jnp.int32),
            pltpu.VMEM((_PER_SUB,), jnp.int32),
            pltpu.VMEM((_PER_SUB,), jnp.int32),
            pltpu.VMEM((_ZSLICE,), jnp.int32),
            pltpu.VMEM_SHARED((APAD,), jnp.int32),
            pltpu.VMEM((_PER_TILE,), jnp.int32),
            pltpu.VMEM((_GCHUNK, H), jnp.float32),
            pltpu.SemaphoreType.DMA,
        ],
    )()
    return fn(topi_flat, rank_flat, off16, h2)


_CPER_TILE = A // SC_TILES
_CCHUNK = _CPER_TILE // 2


def _cgather_body(dest_hbm, eo_hbm, rk_hbm, idx_v, rows_v, sem):
    cid = lax.axis_index("c")
    sid = lax.axis_index("s")
    wid = sid * SC_CORES + cid
    base = wid * _CPER_TILE
    pltpu.sync_copy(dest_hbm.at[pl.ds(base, _CPER_TILE)], idx_v)
    for ch in range(_CPER_TILE // _CCHUNK):
        pltpu.async_copy(eo_hbm.at[idx_v.at[pl.ds(ch * _CCHUNK, _CCHUNK)]],
                         rows_v, sem).wait()
        pltpu.sync_copy(rows_v, rk_hbm.at[pl.ds(base + ch * _CCHUNK, _CCHUNK)])


def _cgather(dest, eo_s):
    mesh = plsc.VectorSubcoreMesh(core_axis_name="c", subcore_axis_name="s")
    fn = functools.partial(
        pl.kernel, _cgather_body, mesh=mesh,
        compiler_params=pltpu.CompilerParams(needs_layout_passes=False),
        out_type=jax.ShapeDtypeStruct((A, H), jnp.float32),
        scratch_types=[
            pltpu.VMEM((_CPER_TILE,), jnp.int32),
            pltpu.VMEM((_CCHUNK, H), jnp.float32),
            pltpu.SemaphoreType.DMA,
        ],
    )()
    return fn(dest, eo_s)


def _rope_tables():
    pos = jnp.arange(S, dtype=jnp.float32)
    inv = 1.0 / (10000.0 ** (jnp.arange(0, DH, 2, dtype=jnp.float32) / DH))
    ang = pos[:, None] * inv[None, :]
    cos = jnp.repeat(jnp.cos(ang), 2, axis=1)
    sin = jnp.repeat(jnp.sin(ang), 2, axis=1)
    return jnp.tile(cos, (1, NH)), jnp.tile(sin, (1, NH))


def kernel(x, attn_norm_g, Wq, Wk, Wv, Wo, ffn_norm_g, router_W, W1, W2):
    x2d = x.reshape(T, H)
    g = attn_norm_g.reshape(1, H)
    g2 = ffn_norm_g.reshape(1, H)
    cos_t, sin_t = _rope_tables()

    q, k, v = _qkv(x2d, g, Wq, Wk, Wv, cos_t, sin_t)
    qh = q.reshape(S, NH, DH).transpose(1, 0, 2)
    kh = k.reshape(S, NH, DH).transpose(1, 0, 2)
    vh = v.reshape(S, NH, DH).transpose(1, 0, 2)
    oh = _attention(qh, kh, vh)
    o2d = oh.transpose(1, 0, 2).reshape(S, H)

    wr_pad = jnp.zeros((H, LANES), jnp.float32).at[:, :E].set(router_W)
    xo, h2, meta, gates, psum, cnts = _router(x2d, o2d, Wo, g2, wr_pad)

    topi_flat = meta[:, :TOPK].reshape(A)
    rank_flat = meta[:, 2:2 + TOPK].reshape(A)
    counts = cnts[0, :E]

    aligned = ((counts.astype(jnp.int32) + BM - 1) // BM) * BM
    po = jnp.cumsum(aligned) - aligned
    off16 = jnp.zeros((16,), jnp.int32).at[:E].set(po)
    chunk_start = jnp.arange(NCHUNK, dtype=jnp.int32) * BM
    e_idx = jnp.sum(chunk_start[:, None] >= po[None, :], axis=1).astype(jnp.int32) - 1

    h2s, dest = _dispatch(topi_flat, rank_flat, off16, h2)
    eo_s = _grouped_ffn(h2s, W1, W2, e_idx)
    rk = _cgather(dest, eo_s)
    out2d = _combine(xo, rk.reshape(S, TOPK, H), gates)

    pmean = psum[0, :E] / T
    frac = counts / T
    aux = (E * jnp.sum(frac * pmean)).astype(jnp.float32)
    return (out2d.reshape(B, S, H), aux)


# speedup vs baseline: 1.1835x; 1.1835x over previous
import functools

import jax
import jax.numpy as jnp
from jax import lax
from jax.experimental import pallas as pl
from jax.experimental.pallas import tpu as pltpu
from jax.experimental.pallas import tpu_sc as plsc

_INTERPRET = False

B, S, H, NH, E, TOPK, FFN = 1, 2048, 1024, 16, 8, 2, 2048
DH = H // NH
T = B * S
A = T * TOPK
BM = 128
APAD = A + E * BM
NCHUNK = APAD // BM
LANES = 128

SC_CORES, SC_SUBCORES = 2, 16
SC_TILES = SC_CORES * SC_SUBCORES

BQ = 256
BS = 256


_BN = 512


def _rope(y, c, s):
    lane = lax.broadcasted_iota(jnp.int32, y.shape, 1)
    even = (lane & 1) == 0
    r_minus = pltpu.roll(y, _BN - 1, 1)
    r_plus = pltpu.roll(y, 1, 1)
    return y * c + jnp.where(even, -r_minus, r_plus) * s


def _qkv_body(x_ref, g_ref, wq_ref, wk_ref, wv_ref,
              cos_ref, sin_ref, q_ref, k_ref, v_ref):
    xv = x_ref[...]
    ms = jnp.mean(xv * xv, axis=-1, keepdims=True)
    h = xv * lax.rsqrt(ms + 1e-6) * g_ref[...]
    c = cos_ref[...]
    s = sin_ref[...]
    q_ref[...] = _rope(jnp.dot(h, wq_ref[...],
                               preferred_element_type=jnp.float32), c, s)
    k_ref[...] = _rope(jnp.dot(h, wk_ref[...],
                               preferred_element_type=jnp.float32), c, s)
    v_ref[...] = jnp.dot(h, wv_ref[...], preferred_element_type=jnp.float32)


def _qkv(x2d, g, wq, wk, wv, cos_t, sin_t):
    grid = (S // BS, H // _BN)
    row = pl.BlockSpec((BS, H), lambda i, j: (i, 0))
    wcol = pl.BlockSpec((H, _BN), lambda i, j: (0, j))
    tile = pl.BlockSpec((BS, _BN), lambda i, j: (i, j))
    gspec = pl.BlockSpec((1, H), lambda i, j: (0, 0))
    return pl.pallas_call(
        _qkv_body,
        grid=grid,
        in_specs=[row, gspec, wcol, wcol, wcol, tile, tile],
        out_specs=[tile, tile, tile],
        out_shape=[jax.ShapeDtypeStruct((S, H), jnp.float32)] * 3,
        interpret=_INTERPRET,
    )(x2d, g, wq, wk, wv, cos_t, sin_t)


def _attn_body(q_ref, k_ref, v_ref, o_ref):
    i = pl.program_id(1)
    q = q_ref[0]
    k = k_ref[0]
    s = lax.dot_general(q, k, (((1,), (1,)), ((), ())),
                        preferred_element_type=jnp.float32)
    s = s * (1.0 / (DH ** 0.5))
    qpos = i * BQ + lax.broadcasted_iota(jnp.int32, (BQ, S), 0)
    kpos = lax.broadcasted_iota(jnp.int32, (BQ, S), 1)
    s = jnp.where(kpos <= qpos, s, -1e9)
    m = jnp.max(s, axis=-1, keepdims=True)
    p = jnp.exp(s - m)
    p = p / jnp.sum(p, axis=-1, keepdims=True)
    o_ref[0] = jnp.dot(p, v_ref[0], preferred_element_type=jnp.float32)


def _attention(qh, kh, vh):
    grid = (NH, S // BQ)
    qspec = pl.BlockSpec((1, BQ, DH), lambda h, i: (h, i, 0))
    kvspec = pl.BlockSpec((1, S, DH), lambda h, i: (h, 0, 0))
    return pl.pallas_call(
        _attn_body,
        grid=grid,
        in_specs=[qspec, kvspec, kvspec],
        out_specs=qspec,
        out_shape=jax.ShapeDtypeStruct((NH, S, DH), jnp.float32),
        interpret=_INTERPRET,
    )(qh, kh, vh)


def _router_body(x_ref, o_ref, wo_ref, g2_ref, wr_ref,
                 xo_ref, h2_ref, meta_ref, gate_ref, psum_ref, cnt_ref):
    i = pl.program_id(0)
    xo = x_ref[...] + jnp.dot(o_ref[...], wo_ref[...],
                              preferred_element_type=jnp.float32)
    xo_ref[...] = xo
    ms = jnp.mean(xo * xo, axis=-1, keepdims=True)
    h2 = xo * lax.rsqrt(ms + 1e-6) * g2_ref[...]
    h2_ref[...] = h2
    lg = jnp.dot(h2, wr_ref[...], preferred_element_type=jnp.float32)
    lane = lax.broadcasted_iota(jnp.int32, (BS, LANES), 1)
    lg = jnp.where(lane < E, lg, -1e30)
    mx = jnp.max(lg, axis=-1, keepdims=True)
    p = jnp.exp(lg - mx)
    p = p / jnp.sum(p, axis=-1, keepdims=True)
    m1 = jnp.max(p, axis=-1, keepdims=True)
    i1 = jnp.min(jnp.where(p == m1, lane, LANES), axis=-1, keepdims=True)
    pm = jnp.where(lane == i1, -1.0, p)
    m2 = jnp.max(pm, axis=-1, keepdims=True)
    i2 = jnp.min(jnp.where(pm == m2, lane, LANES), axis=-1, keepdims=True)
    den = m1 + m2
    g0 = m1 / den
    g1 = m2 / den
    onehot = (lane == i1).astype(jnp.float32) + (lane == i2).astype(jnp.float32)
    rr = lax.broadcasted_iota(jnp.int32, (BS, BS), 0)
    cc = lax.broadcasted_iota(jnp.int32, (BS, BS), 1)
    tril = (cc <= rr).astype(jnp.float32)
    cum = jnp.dot(tril, onehot, preferred_element_type=jnp.float32)

    @pl.when(i == 0)
    def _():
        psum_ref[...] = jnp.zeros_like(psum_ref)
        cnt_ref[...] = jnp.zeros_like(cnt_ref)

    base = cnt_ref[...]
    tot = cum + base
    rank0 = jnp.sum(jnp.where(lane == i1, tot, 0.0), axis=-1, keepdims=True) - 1.0
    rank1 = jnp.sum(jnp.where(lane == i2, tot, 0.0), axis=-1, keepdims=True) - 1.0
    cnt_ref[...] = base + cum[BS - 1:BS, :]
    psum_ref[...] = psum_ref[...] + jnp.sum(p, axis=0, keepdims=True)
    meta = jnp.where(lane == 0, i1,
           jnp.where(lane == 1, i2,
           jnp.where(lane == 2, rank0.astype(jnp.int32),
           jnp.where(lane == 3, rank1.astype(jnp.int32), 0))))
    meta_ref[...] = meta
    gate_ref[...] = jnp.where(lane == 0, g0,
                    jnp.where(lane == 1, g1, 0.0))


def _router(x2d, attn_o, wo, g2, wr_pad):
    grid = (S // BS,)
    row = pl.BlockSpec((BS, H), lambda i: (i, 0))
    meta = pl.BlockSpec((BS, LANES), lambda i: (i, 0))
    acc = pl.BlockSpec((1, LANES), lambda i: (0, 0))
    return pl.pallas_call(
        _router_body,
        grid=grid,
        in_specs=[row, row,
                  pl.BlockSpec((H, H), lambda i: (0, 0)),
                  pl.BlockSpec((1, H), lambda i: (0, 0)),
                  pl.BlockSpec((H, LANES), lambda i: (0, 0))],
        out_specs=[row, row, meta, meta, acc, acc],
        out_shape=[jax.ShapeDtypeStruct((S, H), jnp.float32),
                   jax.ShapeDtypeStruct((S, H), jnp.float32),
                   jax.ShapeDtypeStruct((S, LANES), jnp.int32),
                   jax.ShapeDtypeStruct((S, LANES), jnp.float32),
                   jax.ShapeDtypeStruct((1, LANES), jnp.float32),
                   jax.ShapeDtypeStruct((1, LANES), jnp.float32)],
        interpret=_INTERPRET,
    )(x2d, attn_o, wo, g2, wr_pad)


def _ffn_body(e_ref, x_ref, w1_ref, w2_ref, o_ref):
    hmid = jnp.dot(x_ref[...], w1_ref[0], preferred_element_type=jnp.float32)
    hmid = hmid * (1.0 / (1.0 + jnp.exp(-hmid)))
    o_ref[...] = jnp.dot(hmid, w2_ref[0], preferred_element_type=jnp.float32)


def _grouped_ffn(h2s, w1, w2, e_idx):
    grid_spec = pltpu.PrefetchScalarGridSpec(
        num_scalar_prefetch=1,
        grid=(NCHUNK,),
        in_specs=[
            pl.BlockSpec((BM, H), lambda i, e_ref: (i, 0)),
            pl.BlockSpec((1, H, FFN), lambda i, e_ref: (e_ref[i], 0, 0)),
            pl.BlockSpec((1, FFN, H), lambda i, e_ref: (e_ref[i], 0, 0)),
        ],
        out_specs=pl.BlockSpec((BM, H), lambda i, e_ref: (i, 0)),
    )
    return pl.pallas_call(
        _ffn_body,
        grid_spec=grid_spec,
        out_shape=jax.ShapeDtypeStruct((APAD, H), jnp.float32),
        interpret=_INTERPRET,
    )(e_idx, h2s, w1, w2)


def _combine_body(xo_ref, rk_ref, gate_ref, out_ref):
    g0 = gate_ref[:, 0:1]
    g1 = gate_ref[:, 1:2]
    out_ref[...] = xo_ref[...] + g0 * rk_ref[:, 0, :] + g1 * rk_ref[:, 1, :]


def _combine(xo, rk3, gates):
    grid = (S // BS,)
    row = pl.BlockSpec((BS, H), lambda i: (i, 0))
    return pl.pallas_call(
        _combine_body,
        grid=grid,
        in_specs=[row,
                  pl.BlockSpec((BS, TOPK, H), lambda i: (i, 0, 0)),
                  pl.BlockSpec((BS, LANES), lambda i: (i, 0))],
        out_specs=row,
        out_shape=jax.ShapeDtypeStruct((S, H), jnp.float32),
        interpret=_INTERPRET,
    )(xo, rk3, gates)


_PER_TILE = APAD // SC_TILES
_GCHUNK = _PER_TILE // 2
_PER_SUB = A // SC_SUBCORES
_ZSLICE = APAD // SC_SUBCORES


def _dispatch_body(topi_hbm, rank_hbm, off_hbm, h2_hbm, h2s_hbm, dest_hbm,
                   topi_v, rank_v, off_v, dest_v, tok_v, zero_v,
                   shared_sort, idx0_v, idx1_v, rows_v, sem):
    idx_bufs = (idx0_v, idx1_v)
    cid = lax.axis_index("c")
    sid = lax.axis_index("s")
    wid = sid * SC_CORES + cid

    abase = sid * _PER_SUB
    pltpu.sync_copy(topi_hbm.at[pl.ds(abase, _PER_SUB)], topi_v)
    pltpu.sync_copy(rank_hbm.at[pl.ds(abase, _PER_SUB)], rank_v)
    pltpu.sync_copy(off_hbm, off_v)

    def zbody(i, carry):
        zero_v[pl.ds(i * 16, 16)] = jnp.zeros((16,), jnp.int32)
        return carry
    lax.fori_loop(0, _ZSLICE // 16, zbody, 0)
    pltpu.sync_copy(zero_v, shared_sort.at[pl.ds(sid * _ZSLICE, _ZSLICE)])

    def body(i, carry):
        e = topi_v[pl.ds(i * 16, 16)]
        r = rank_v[pl.ds(i * 16, 16)]
        d = plsc.load_gather(off_v, [e]) + r
        dest_v[pl.ds(i * 16, 16)] = d
        a = abase + i * 16 + lax.iota(jnp.int32, 16)
        tok_v[pl.ds(i * 16, 16)] = a >> 1
        return carry
    lax.fori_loop(0, _PER_SUB // 16, body, 0)

    @pl.when(cid == 0)
    def _():
        pltpu.sync_copy(dest_v, dest_hbm.at[pl.ds(abase, _PER_SUB)])

    plsc.subcore_barrier()
    pltpu.sync_copy(tok_v, shared_sort.at[dest_v], add=True)
    plsc.subcore_barrier()

    base = wid * _PER_TILE
    for ch, idx_v in enumerate(idx_bufs):
        pltpu.sync_copy(shared_sort.at[pl.ds(base + ch * _GCHUNK, _GCHUNK)],
                        idx_v)
        pltpu.async_copy(h2_hbm.at[idx_v], rows_v, sem).wait()
        pltpu.sync_copy(rows_v, h2s_hbm.at[pl.ds(base + ch * _GCHUNK, _GCHUNK)])


def _dispatch(topi_flat, rank_flat, off16, h2):
    mesh = plsc.VectorSubcoreMesh(core_axis_name="c", subcore_axis_name="s")
    fn = functools.partial(
        pl.kernel, _dispatch_body, mesh=mesh,
        compiler_params=pltpu.CompilerParams(needs_layout_passes=False, use_tc_tiling_on_sc=True),
        out_type=[jax.ShapeDtypeStruct((APAD, H), jnp.float32),
                  jax.ShapeDtypeStruct((A,), jnp.int32)],
        scratch_types=[
            pltpu.VMEM((_PER_SUB,), jnp.int32),
            pltpu.VMEM((_PER_SUB,), jnp.int32),
            pltpu.VMEM((16,), jnp.int32),
            pltpu.VMEM((_PER_SUB,), jnp.int32),
            pltpu.VMEM((_PER_SUB,), jnp.int32),
            pltpu.VMEM((_ZSLICE,), jnp.int32),
            pltpu.VMEM_SHARED((APAD,), jnp.int32),
            pltpu.VMEM((_GCHUNK,), jnp.int32),
            pltpu.VMEM((_GCHUNK,), jnp.int32),
            pltpu.VMEM((_GCHUNK, H), jnp.float32),
            pltpu.SemaphoreType.DMA,
        ],
    )()
    return fn(topi_flat, rank_flat, off16, h2)


_CPER_TILE = A // SC_TILES
_CCHUNK = _CPER_TILE // 2


def _cgather_body(dest_hbm, eo_hbm, rk_hbm, idx0_v, idx1_v, rows_v, sem):
    cid = lax.axis_index("c")
    sid = lax.axis_index("s")
    wid = sid * SC_CORES + cid
    base = wid * _CPER_TILE
    for ch, idx_v in enumerate((idx0_v, idx1_v)):
        pltpu.sync_copy(dest_hbm.at[pl.ds(base + ch * _CCHUNK, _CCHUNK)], idx_v)
        pltpu.async_copy(eo_hbm.at[idx_v], rows_v, sem).wait()
        pltpu.sync_copy(rows_v, rk_hbm.at[pl.ds(base + ch * _CCHUNK, _CCHUNK)])


def _cgather(dest, eo_s):
    mesh = plsc.VectorSubcoreMesh(core_axis_name="c", subcore_axis_name="s")
    fn = functools.partial(
        pl.kernel, _cgather_body, mesh=mesh,
        compiler_params=pltpu.CompilerParams(needs_layout_passes=False, use_tc_tiling_on_sc=True),
        out_type=jax.ShapeDtypeStruct((A, H), jnp.float32),
        scratch_types=[
            pltpu.VMEM((_CCHUNK,), jnp.int32),
            pltpu.VMEM((_CCHUNK,), jnp.int32),
            pltpu.VMEM((_CCHUNK, H), jnp.float32),
            pltpu.SemaphoreType.DMA,
        ],
    )()
    return fn(dest, eo_s)


def _rope_tables():
    pos = jnp.arange(S, dtype=jnp.float32)
    inv = 1.0 / (10000.0 ** (jnp.arange(0, DH, 2, dtype=jnp.float32) / DH))
    ang = pos[:, None] * inv[None, :]
    cos = jnp.repeat(jnp.cos(ang), 2, axis=1)
    sin = jnp.repeat(jnp.sin(ang), 2, axis=1)
    return jnp.tile(cos, (1, NH)), jnp.tile(sin, (1, NH))


def kernel(x, attn_norm_g, Wq, Wk, Wv, Wo, ffn_norm_g, router_W, W1, W2):
    x2d = x.reshape(T, H)
    g = attn_norm_g.reshape(1, H)
    g2 = ffn_norm_g.reshape(1, H)
    cos_t, sin_t = _rope_tables()

    q, k, v = _qkv(x2d, g, Wq, Wk, Wv, cos_t, sin_t)
    qh = q.reshape(S, NH, DH).transpose(1, 0, 2)
    kh = k.reshape(S, NH, DH).transpose(1, 0, 2)
    vh = v.reshape(S, NH, DH).transpose(1, 0, 2)
    oh = _attention(qh, kh, vh)
    o2d = oh.transpose(1, 0, 2).reshape(S, H)

    wr_pad = jnp.zeros((H, LANES), jnp.float32).at[:, :E].set(router_W)
    xo, h2, meta, gates, psum, cnts = _router(x2d, o2d, Wo, g2, wr_pad)

    topi_flat = meta[:, :TOPK].reshape(A)
    rank_flat = meta[:, 2:2 + TOPK].reshape(A)
    counts = cnts[0, :E]

    aligned = ((counts.astype(jnp.int32) + BM - 1) // BM) * BM
    po = jnp.cumsum(aligned) - aligned
    off16 = jnp.zeros((16,), jnp.int32).at[:E].set(po)
    chunk_start = jnp.arange(NCHUNK, dtype=jnp.int32) * BM
    e_idx = jnp.sum(chunk_start[:, None] >= po[None, :], axis=1).astype(jnp.int32) - 1

    h2s, dest = _dispatch(topi_flat, rank_flat, off16, h2)
    eo_s = _grouped_ffn(h2s, W1, W2, e_idx)
    rk = _cgather(dest, eo_s)
    out2d = _combine(xo, rk.reshape(S, TOPK, H), gates)

    pmean = psum[0, :E] / T
    frac = counts / T
    aux = (E * jnp.sum(frac * pmean)).astype(jnp.float32)
    return (out2d.reshape(B, S, H), aux)


# speedup vs baseline: 1.2121x; 1.0242x over previous
import functools

import jax
import jax.numpy as jnp
from jax import lax
from jax.experimental import pallas as pl
from jax.experimental.pallas import tpu as pltpu
from jax.experimental.pallas import tpu_sc as plsc

_INTERPRET = False

B, S, H, NH, E, TOPK, FFN = 1, 2048, 1024, 16, 8, 2, 2048
DH = H // NH
T = B * S
A = T * TOPK
BM = 128
APAD = A + E * BM
NCHUNK = APAD // BM
LANES = 128

SC_CORES, SC_SUBCORES = 2, 16
SC_TILES = SC_CORES * SC_SUBCORES

BQ = 256
BS = 256


_BN = 512


def _rope(y, c, s):
    lane = lax.broadcasted_iota(jnp.int32, y.shape, 1)
    even = (lane & 1) == 0
    r_minus = pltpu.roll(y, _BN - 1, 1)
    r_plus = pltpu.roll(y, 1, 1)
    return y * c + jnp.where(even, -r_minus, r_plus) * s


def _qkv_body(x_ref, g_ref, wq_ref, wk_ref, wv_ref,
              cos_ref, sin_ref, q_ref, k_ref, v_ref):
    xv = x_ref[...]
    ms = jnp.mean(xv * xv, axis=-1, keepdims=True)
    h = xv * lax.rsqrt(ms + 1e-6) * g_ref[...]
    c = cos_ref[...]
    s = sin_ref[...]
    q_ref[...] = _rope(jnp.dot(h, wq_ref[...],
                               preferred_element_type=jnp.float32), c, s)
    k_ref[...] = _rope(jnp.dot(h, wk_ref[...],
                               preferred_element_type=jnp.float32), c, s)
    v_ref[...] = jnp.dot(h, wv_ref[...], preferred_element_type=jnp.float32)


def _qkv(x2d, g, wq, wk, wv, cos_t, sin_t):
    grid = (S // BS, H // _BN)
    row = pl.BlockSpec((BS, H), lambda i, j: (i, 0))
    wcol = pl.BlockSpec((H, _BN), lambda i, j: (0, j))
    tile = pl.BlockSpec((BS, _BN), lambda i, j: (i, j))
    gspec = pl.BlockSpec((1, H), lambda i, j: (0, 0))
    return pl.pallas_call(
        _qkv_body,
        grid=grid,
        in_specs=[row, gspec, wcol, wcol, wcol, tile, tile],
        out_specs=[tile, tile, tile],
        out_shape=[jax.ShapeDtypeStruct((S, H), jnp.float32)] * 3,
        interpret=_INTERPRET,
    )(x2d, g, wq, wk, wv, cos_t, sin_t)


def _attn_body(q_ref, k_ref, v_ref, o_ref):
    i = pl.program_id(1)
    q = q_ref[0] * (1.0 / (DH ** 0.5))
    k = k_ref[0]
    s = lax.dot_general(q, k, (((1,), (1,)), ((), ())),
                        preferred_element_type=jnp.float32)
    qpos = i * BQ + lax.broadcasted_iota(jnp.int32, (BQ, S), 0)
    kpos = lax.broadcasted_iota(jnp.int32, (BQ, S), 1)
    s = jnp.where(kpos <= qpos, s, -1e9)
    m = jnp.max(s, axis=-1, keepdims=True)
    p = jnp.exp(s - m)
    l = jnp.sum(p, axis=-1, keepdims=True)
    o = jnp.dot(p, v_ref[0], preferred_element_type=jnp.float32)
    o_ref[0] = o / l


def _attention(qh, kh, vh):
    grid = (NH, S // BQ)
    qspec = pl.BlockSpec((1, BQ, DH), lambda h, i: (h, i, 0))
    kvspec = pl.BlockSpec((1, S, DH), lambda h, i: (h, 0, 0))
    return pl.pallas_call(
        _attn_body,
        grid=grid,
        in_specs=[qspec, kvspec, kvspec],
        out_specs=qspec,
        out_shape=jax.ShapeDtypeStruct((NH, S, DH), jnp.float32),
        interpret=_INTERPRET,
    )(qh, kh, vh)


def _router_body(x_ref, o_ref, wo_ref, g2_ref, wr_ref,
                 xo_ref, h2_ref, meta_ref, gate_ref, psum_ref, cnt_ref):
    i = pl.program_id(0)
    xo = x_ref[...] + jnp.dot(o_ref[...], wo_ref[...],
                              preferred_element_type=jnp.float32)
    xo_ref[...] = xo
    ms = jnp.mean(xo * xo, axis=-1, keepdims=True)
    h2 = xo * lax.rsqrt(ms + 1e-6) * g2_ref[...]
    h2_ref[...] = h2
    lg = jnp.dot(h2, wr_ref[...], preferred_element_type=jnp.float32)
    lane = lax.broadcasted_iota(jnp.int32, (BS, LANES), 1)
    lg = jnp.where(lane < E, lg, -1e30)
    mx = jnp.max(lg, axis=-1, keepdims=True)
    p = jnp.exp(lg - mx)
    p = p / jnp.sum(p, axis=-1, keepdims=True)
    m1 = jnp.max(p, axis=-1, keepdims=True)
    i1 = jnp.min(jnp.where(p == m1, lane, LANES), axis=-1, keepdims=True)
    pm = jnp.where(lane == i1, -1.0, p)
    m2 = jnp.max(pm, axis=-1, keepdims=True)
    i2 = jnp.min(jnp.where(pm == m2, lane, LANES), axis=-1, keepdims=True)
    den = m1 + m2
    g0 = m1 / den
    g1 = m2 / den
    onehot = (lane == i1).astype(jnp.float32) + (lane == i2).astype(jnp.float32)
    rr = lax.broadcasted_iota(jnp.int32, (BS, BS), 0)
    cc = lax.broadcasted_iota(jnp.int32, (BS, BS), 1)
    tril = (cc <= rr).astype(jnp.float32)
    cum = jnp.dot(tril, onehot, preferred_element_type=jnp.float32)

    @pl.when(i == 0)
    def _():
        psum_ref[...] = jnp.zeros_like(psum_ref)
        cnt_ref[...] = jnp.zeros_like(cnt_ref)

    base = cnt_ref[...]
    tot = cum + base
    rank0 = jnp.sum(jnp.where(lane == i1, tot, 0.0), axis=-1, keepdims=True) - 1.0
    rank1 = jnp.sum(jnp.where(lane == i2, tot, 0.0), axis=-1, keepdims=True) - 1.0
    cnt_ref[...] = base + cum[BS - 1:BS, :]
    psum_ref[...] = psum_ref[...] + jnp.sum(p, axis=0, keepdims=True)
    meta = jnp.where(lane == 0, i1,
           jnp.where(lane == 1, i2,
           jnp.where(lane == 2, rank0.astype(jnp.int32),
           jnp.where(lane == 3, rank1.astype(jnp.int32), 0))))
    meta_ref[...] = meta
    gate_ref[...] = jnp.where(lane == 0, g0,
                    jnp.where(lane == 1, g1, 0.0))


def _router(x2d, attn_o, wo, g2, wr_pad):
    grid = (S // BS,)
    row = pl.BlockSpec((BS, H), lambda i: (i, 0))
    meta = pl.BlockSpec((BS, LANES), lambda i: (i, 0))
    acc = pl.BlockSpec((1, LANES), lambda i: (0, 0))
    return pl.pallas_call(
        _router_body,
        grid=grid,
        in_specs=[row, row,
                  pl.BlockSpec((H, H), lambda i: (0, 0)),
                  pl.BlockSpec((1, H), lambda i: (0, 0)),
                  pl.BlockSpec((H, LANES), lambda i: (0, 0))],
        out_specs=[row, row, meta, meta, acc, acc],
        out_shape=[jax.ShapeDtypeStruct((S, H), jnp.float32),
                   jax.ShapeDtypeStruct((S, H), jnp.float32),
                   jax.ShapeDtypeStruct((S, LANES), jnp.int32),
                   jax.ShapeDtypeStruct((S, LANES), jnp.float32),
                   jax.ShapeDtypeStruct((1, LANES), jnp.float32),
                   jax.ShapeDtypeStruct((1, LANES), jnp.float32)],
        interpret=_INTERPRET,
    )(x2d, attn_o, wo, g2, wr_pad)


def _ffn_body(e_ref, x_ref, w1_ref, w2_ref, o_ref):
    hmid = jnp.dot(x_ref[...], w1_ref[0], preferred_element_type=jnp.float32)
    hmid = hmid * (1.0 / (1.0 + jnp.exp(-hmid)))
    o_ref[...] = jnp.dot(hmid, w2_ref[0], preferred_element_type=jnp.float32)


def _grouped_ffn(h2s, w1, w2, e_idx):
    grid_spec = pltpu.PrefetchScalarGridSpec(
        num_scalar_prefetch=1,
        grid=(NCHUNK,),
        in_specs=[
            pl.BlockSpec((BM, H), lambda i, e_ref: (i, 0)),
            pl.BlockSpec((1, H, FFN), lambda i, e_ref: (e_ref[i], 0, 0)),
            pl.BlockSpec((1, FFN, H), lambda i, e_ref: (e_ref[i], 0, 0)),
        ],
        out_specs=pl.BlockSpec((BM, H), lambda i, e_ref: (i, 0)),
    )
    return pl.pallas_call(
        _ffn_body,
        grid_spec=grid_spec,
        out_shape=jax.ShapeDtypeStruct((APAD, H), jnp.float32),
        interpret=_INTERPRET,
    )(e_idx, h2s, w1, w2)


def _combine_body(xo_ref, rk_ref, gate_ref, out_ref):
    g0 = gate_ref[:, 0:1]
    g1 = gate_ref[:, 1:2]
    out_ref[...] = xo_ref[...] + g0 * rk_ref[:, 0, :] + g1 * rk_ref[:, 1, :]


def _combine(xo, rk3, gates):
    grid = (S // BS,)
    row = pl.BlockSpec((BS, H), lambda i: (i, 0))
    return pl.pallas_call(
        _combine_body,
        grid=grid,
        in_specs=[row,
                  pl.BlockSpec((BS, TOPK, H), lambda i: (i, 0, 0)),
                  pl.BlockSpec((BS, LANES), lambda i: (i, 0))],
        out_specs=row,
        out_shape=jax.ShapeDtypeStruct((S, H), jnp.float32),
        interpret=_INTERPRET,
    )(xo, rk3, gates)


_PER_TILE = APAD // SC_TILES
_GCHUNK = _PER_TILE // 2
_PER_SUB = A // SC_SUBCORES
_ZSLICE = APAD // SC_SUBCORES


def _dispatch_body(topi_hbm, rank_hbm, off_hbm, h2_hbm, h2s_hbm, dest_hbm,
                   topi_v, rank_v, off_v, dest_v, tok_v, zero_v,
                   shared_sort, idx0_v, idx1_v, rows_v, sem):
    idx_bufs = (idx0_v, idx1_v)
    cid = lax.axis_index("c")
    sid = lax.axis_index("s")
    wid = sid * SC_CORES + cid

    abase = sid * _PER_SUB
    pltpu.sync_copy(topi_hbm.at[pl.ds(abase, _PER_SUB)], topi_v)
    pltpu.sync_copy(rank_hbm.at[pl.ds(abase, _PER_SUB)], rank_v)
    pltpu.sync_copy(off_hbm, off_v)

    def zbody(i, carry):
        zero_v[pl.ds(i * 16, 16)] = jnp.zeros((16,), jnp.int32)
        return carry
    lax.fori_loop(0, _ZSLICE // 16, zbody, 0)
    pltpu.sync_copy(zero_v, shared_sort.at[pl.ds(sid * _ZSLICE, _ZSLICE)])

    def body(i, carry):
        e = topi_v[pl.ds(i * 16, 16)]
        r = rank_v[pl.ds(i * 16, 16)]
        d = plsc.load_gather(off_v, [e]) + r
        dest_v[pl.ds(i * 16, 16)] = d
        a = abase + i * 16 + lax.iota(jnp.int32, 16)
        tok_v[pl.ds(i * 16, 16)] = a >> 1
        return carry
    lax.fori_loop(0, _PER_SUB // 16, body, 0)

    @pl.when(cid == 0)
    def _():
        pltpu.sync_copy(dest_v, dest_hbm.at[pl.ds(abase, _PER_SUB)])

    plsc.subcore_barrier()
    pltpu.sync_copy(tok_v, shared_sort.at[dest_v], add=True)
    plsc.subcore_barrier()

    base = wid * _PER_TILE
    for ch, idx_v in enumerate(idx_bufs):
        pltpu.sync_copy(shared_sort.at[pl.ds(base + ch * _GCHUNK, _GCHUNK)],
                        idx_v)
        pltpu.async_copy(h2_hbm.at[idx_v], rows_v, sem).wait()
        pltpu.sync_copy(rows_v, h2s_hbm.at[pl.ds(base + ch * _GCHUNK, _GCHUNK)])


def _dispatch(topi_flat, rank_flat, off16, h2):
    mesh = plsc.VectorSubcoreMesh(core_axis_name="c", subcore_axis_name="s")
    fn = functools.partial(
        pl.kernel, _dispatch_body, mesh=mesh,
        compiler_params=pltpu.CompilerParams(needs_layout_passes=False, use_tc_tiling_on_sc=True),
        out_type=[jax.ShapeDtypeStruct((APAD, H), jnp.float32),
                  jax.ShapeDtypeStruct((A,), jnp.int32)],
        scratch_types=[
            pltpu.VMEM((_PER_SUB,), jnp.int32),
            pltpu.VMEM((_PER_SUB,), jnp.int32),
            pltpu.VMEM((16,), jnp.int32),
            pltpu.VMEM((_PER_SUB,), jnp.int32),
            pltpu.VMEM((_PER_SUB,), jnp.int32),
            pltpu.VMEM((_ZSLICE,), jnp.int32),
            pltpu.VMEM_SHARED((APAD,), jnp.int32),
            pltpu.VMEM((_GCHUNK,), jnp.int32),
            pltpu.VMEM((_GCHUNK,), jnp.int32),
            pltpu.VMEM((_GCHUNK, H), jnp.float32),
            pltpu.SemaphoreType.DMA,
        ],
    )()
    return fn(topi_flat, rank_flat, off16, h2)


_CPER_TILE = A // SC_TILES
_CCHUNK = _CPER_TILE // 2


def _cgather_body(dest_hbm, eo_hbm, rk_hbm, idx0_v, idx1_v, rows_v, sem):
    cid = lax.axis_index("c")
    sid = lax.axis_index("s")
    wid = sid * SC_CORES + cid
    base = wid * _CPER_TILE
    for ch, idx_v in enumerate((idx0_v, idx1_v)):
        pltpu.sync_copy(dest_hbm.at[pl.ds(base + ch * _CCHUNK, _CCHUNK)], idx_v)
        pltpu.async_copy(eo_hbm.at[idx_v], rows_v, sem).wait()
        pltpu.sync_copy(rows_v, rk_hbm.at[pl.ds(base + ch * _CCHUNK, _CCHUNK)])


def _cgather(dest, eo_s):
    mesh = plsc.VectorSubcoreMesh(core_axis_name="c", subcore_axis_name="s")
    fn = functools.partial(
        pl.kernel, _cgather_body, mesh=mesh,
        compiler_params=pltpu.CompilerParams(needs_layout_passes=False, use_tc_tiling_on_sc=True),
        out_type=jax.ShapeDtypeStruct((A, H), jnp.float32),
        scratch_types=[
            pltpu.VMEM((_CCHUNK,), jnp.int32),
            pltpu.VMEM((_CCHUNK,), jnp.int32),
            pltpu.VMEM((_CCHUNK, H), jnp.float32),
            pltpu.SemaphoreType.DMA,
        ],
    )()
    return fn(dest, eo_s)


def _rope_tables():
    pos = jnp.arange(S, dtype=jnp.float32)
    inv = 1.0 / (10000.0 ** (jnp.arange(0, DH, 2, dtype=jnp.float32) / DH))
    ang = pos[:, None] * inv[None, :]
    cos = jnp.repeat(jnp.cos(ang), 2, axis=1)
    sin = jnp.repeat(jnp.sin(ang), 2, axis=1)
    return jnp.tile(cos, (1, NH)), jnp.tile(sin, (1, NH))


def kernel(x, attn_norm_g, Wq, Wk, Wv, Wo, ffn_norm_g, router_W, W1, W2):
    x2d = x.reshape(T, H)
    g = attn_norm_g.reshape(1, H)
    g2 = ffn_norm_g.reshape(1, H)
    cos_t, sin_t = _rope_tables()

    q, k, v = _qkv(x2d, g, Wq, Wk, Wv, cos_t, sin_t)
    qh = q.reshape(S, NH, DH).transpose(1, 0, 2)
    kh = k.reshape(S, NH, DH).transpose(1, 0, 2)
    vh = v.reshape(S, NH, DH).transpose(1, 0, 2)
    oh = _attention(qh, kh, vh)
    o2d = oh.transpose(1, 0, 2).reshape(S, H)

    wr_pad = jnp.zeros((H, LANES), jnp.float32).at[:, :E].set(router_W)
    xo, h2, meta, gates, psum, cnts = _router(x2d, o2d, Wo, g2, wr_pad)

    topi_flat = meta[:, :TOPK].reshape(A)
    rank_flat = meta[:, 2:2 + TOPK].reshape(A)
    counts = cnts[0, :E]

    aligned = ((counts.astype(jnp.int32) + BM - 1) // BM) * BM
    po = jnp.cumsum(aligned) - aligned
    off16 = jnp.zeros((16,), jnp.int32).at[:E].set(po)
    chunk_start = jnp.arange(NCHUNK, dtype=jnp.int32) * BM
    e_idx = jnp.sum(chunk_start[:, None] >= po[None, :], axis=1).astype(jnp.int32) - 1

    h2s, dest = _dispatch(topi_flat, rank_flat, off16, h2)
    eo_s = _grouped_ffn(h2s, W1, W2, e_idx)
    rk = _cgather(dest, eo_s)
    out2d = _combine(xo, rk.reshape(S, TOPK, H), gates)

    pmean = psum[0, :E] / T
    frac = counts / T
    aux = (E * jnp.sum(frac * pmean)).astype(jnp.float32)
    return (out2d.reshape(B, S, H), aux)


# speedup vs baseline: 1.3435x; 1.1084x over previous
import functools

import jax
import jax.numpy as jnp
from jax import lax
from jax.experimental import pallas as pl
from jax.experimental.pallas import tpu as pltpu
from jax.experimental.pallas import tpu_sc as plsc

_INTERPRET = False

B, S, H, NH, E, TOPK, FFN = 1, 2048, 1024, 16, 8, 2, 2048
DH = H // NH
T = B * S
A = T * TOPK
BM = 128
APAD = A + E * BM
NCHUNK = APAD // BM
LANES = 128

SC_CORES, SC_SUBCORES = 2, 16
SC_TILES = SC_CORES * SC_SUBCORES

BQ = 256
BS = 256


_BN = 512


def _rope(y, c, s):
    lane = lax.broadcasted_iota(jnp.int32, y.shape, 1)
    even = (lane & 1) == 0
    r_minus = pltpu.roll(y, _BN - 1, 1)
    r_plus = pltpu.roll(y, 1, 1)
    return y * c + jnp.where(even, -r_minus, r_plus) * s


def _qkv_body(x_ref, g_ref, wq_ref, wk_ref, wv_ref,
              cos_ref, sin_ref, q_ref, k_ref, v_ref):
    xv = x_ref[...]
    ms = jnp.mean(xv * xv, axis=-1, keepdims=True)
    h = xv * lax.rsqrt(ms + 1e-6) * g_ref[...]
    c = cos_ref[...]
    s = sin_ref[...]
    q_ref[...] = _rope(jnp.dot(h, wq_ref[...],
                               preferred_element_type=jnp.float32), c, s)
    k_ref[...] = _rope(jnp.dot(h, wk_ref[...],
                               preferred_element_type=jnp.float32), c, s)
    v_ref[...] = jnp.dot(h, wv_ref[...], preferred_element_type=jnp.float32)


def _qkv(x2d, g, wq, wk, wv, cos_t, sin_t):
    grid = (S // BS, H // _BN)
    row = pl.BlockSpec((BS, H), lambda i, j: (i, 0))
    wcol = pl.BlockSpec((H, _BN), lambda i, j: (0, j))
    tile = pl.BlockSpec((BS, _BN), lambda i, j: (i, j))
    gspec = pl.BlockSpec((1, H), lambda i, j: (0, 0))
    return pl.pallas_call(
        _qkv_body,
        grid=grid,
        in_specs=[row, gspec, wcol, wcol, wcol, tile, tile],
        out_specs=[tile, tile, tile],
        out_shape=[jax.ShapeDtypeStruct((S, H), jnp.float32)] * 3,
        interpret=_INTERPRET,
    )(x2d, g, wq, wk, wv, cos_t, sin_t)


def _attn_body(q_ref, k_ref, v_ref, o_ref):
    i = pl.program_id(1)
    q = q_ref[0] * (1.0 / (DH ** 0.5))
    k = k_ref[0]
    s = lax.dot_general(q, k, (((1,), (1,)), ((), ())),
                        preferred_element_type=jnp.float32)
    qpos = i * BQ + lax.broadcasted_iota(jnp.int32, (BQ, S), 0)
    kpos = lax.broadcasted_iota(jnp.int32, (BQ, S), 1)
    s = jnp.where(kpos <= qpos, s, -1e9)
    m = jnp.max(s, axis=-1, keepdims=True)
    p = jnp.exp(s - m)
    l = jnp.sum(p, axis=-1, keepdims=True)
    o = jnp.dot(p, v_ref[0], preferred_element_type=jnp.float32)
    o_ref[0] = o / l


def _attention(qh, kh, vh):
    grid = (NH, S // BQ)
    qspec = pl.BlockSpec((1, BQ, DH), lambda h, i: (h, i, 0))
    kvspec = pl.BlockSpec((1, S, DH), lambda h, i: (h, 0, 0))
    return pl.pallas_call(
        _attn_body,
        grid=grid,
        in_specs=[qspec, kvspec, kvspec],
        out_specs=qspec,
        out_shape=jax.ShapeDtypeStruct((NH, S, DH), jnp.float32),
        interpret=_INTERPRET,
    )(qh, kh, vh)


def _router_body(x_ref, o_ref, wo_ref, g2_ref, wr_ref,
                 xo_ref, h2_ref, meta_ref, gate_ref, psum_ref, cnt_ref):
    i = pl.program_id(0)
    xo = x_ref[...] + jnp.dot(o_ref[...], wo_ref[...],
                              preferred_element_type=jnp.float32)
    xo_ref[...] = xo
    ms = jnp.mean(xo * xo, axis=-1, keepdims=True)
    h2 = xo * lax.rsqrt(ms + 1e-6) * g2_ref[...]
    h2_ref[...] = h2
    lg = jnp.dot(h2, wr_ref[...], preferred_element_type=jnp.float32)
    lane = lax.broadcasted_iota(jnp.int32, (BS, LANES), 1)
    lg = jnp.where(lane < E, lg, -1e30)
    mx = jnp.max(lg, axis=-1, keepdims=True)
    p = jnp.exp(lg - mx)
    p = p / jnp.sum(p, axis=-1, keepdims=True)
    m1 = jnp.max(p, axis=-1, keepdims=True)
    i1 = jnp.min(jnp.where(p == m1, lane, LANES), axis=-1, keepdims=True)
    pm = jnp.where(lane == i1, -1.0, p)
    m2 = jnp.max(pm, axis=-1, keepdims=True)
    i2 = jnp.min(jnp.where(pm == m2, lane, LANES), axis=-1, keepdims=True)
    den = m1 + m2
    g0 = m1 / den
    g1 = m2 / den
    onehot = (lane == i1).astype(jnp.float32) + (lane == i2).astype(jnp.float32)
    rr = lax.broadcasted_iota(jnp.int32, (BS, BS), 0)
    cc = lax.broadcasted_iota(jnp.int32, (BS, BS), 1)
    tril = (cc <= rr).astype(jnp.float32)
    cum = jnp.dot(tril, onehot, preferred_element_type=jnp.float32)

    @pl.when(i == 0)
    def _():
        psum_ref[...] = jnp.zeros_like(psum_ref)
        cnt_ref[...] = jnp.zeros_like(cnt_ref)

    base = cnt_ref[...]
    tot = cum + base
    rank0 = jnp.sum(jnp.where(lane == i1, tot, 0.0), axis=-1, keepdims=True) - 1.0
    rank1 = jnp.sum(jnp.where(lane == i2, tot, 0.0), axis=-1, keepdims=True) - 1.0
    cnt_ref[...] = base + cum[BS - 1:BS, :]
    psum_ref[...] = psum_ref[...] + jnp.sum(p, axis=0, keepdims=True)
    meta = jnp.where(lane == 0, i1,
           jnp.where(lane == 1, i2,
           jnp.where(lane == 2, rank0.astype(jnp.int32),
           jnp.where(lane == 3, rank1.astype(jnp.int32), 0))))
    meta_ref[...] = meta
    gate_ref[...] = jnp.where(lane == 0, g0,
                    jnp.where(lane == 1, g1, 0.0))


def _router(x2d, attn_o, wo, g2, wr_pad):
    grid = (S // BS,)
    row = pl.BlockSpec((BS, H), lambda i: (i, 0))
    meta = pl.BlockSpec((BS, LANES), lambda i: (i, 0))
    acc = pl.BlockSpec((1, LANES), lambda i: (0, 0))
    return pl.pallas_call(
        _router_body,
        grid=grid,
        in_specs=[row, row,
                  pl.BlockSpec((H, H), lambda i: (0, 0)),
                  pl.BlockSpec((1, H), lambda i: (0, 0)),
                  pl.BlockSpec((H, LANES), lambda i: (0, 0))],
        out_specs=[row, row, meta, meta, acc, acc],
        out_shape=[jax.ShapeDtypeStruct((S, H), jnp.float32),
                   jax.ShapeDtypeStruct((S, H), jnp.float32),
                   jax.ShapeDtypeStruct((S, LANES), jnp.int32),
                   jax.ShapeDtypeStruct((S, LANES), jnp.float32),
                   jax.ShapeDtypeStruct((1, LANES), jnp.float32),
                   jax.ShapeDtypeStruct((1, LANES), jnp.float32)],
        interpret=_INTERPRET,
    )(x2d, attn_o, wo, g2, wr_pad)


def _ffn_body(e_ref, x_ref, w1_ref, w2_ref, o_ref):
    hmid = jnp.dot(x_ref[...], w1_ref[0], preferred_element_type=jnp.float32)
    hmid = hmid * (1.0 / (1.0 + jnp.exp(-hmid)))
    o_ref[...] = jnp.dot(hmid, w2_ref[0], preferred_element_type=jnp.float32)


def _grouped_ffn(h2s, w1, w2, e_idx):
    grid_spec = pltpu.PrefetchScalarGridSpec(
        num_scalar_prefetch=1,
        grid=(NCHUNK,),
        in_specs=[
            pl.BlockSpec((BM, H), lambda i, e_ref: (i, 0)),
            pl.BlockSpec((1, H, FFN), lambda i, e_ref: (e_ref[i], 0, 0)),
            pl.BlockSpec((1, FFN, H), lambda i, e_ref: (e_ref[i], 0, 0)),
        ],
        out_specs=pl.BlockSpec((BM, H), lambda i, e_ref: (i, 0)),
    )
    return pl.pallas_call(
        _ffn_body,
        grid_spec=grid_spec,
        out_shape=jax.ShapeDtypeStruct((APAD, H), jnp.float32),
        interpret=_INTERPRET,
    )(e_idx, h2s, w1, w2)


def _combine_body(xo_ref, rk_ref, gate_ref, out_ref):
    g0 = gate_ref[:, 0:1]
    g1 = gate_ref[:, 1:2]
    out_ref[...] = xo_ref[...] + g0 * rk_ref[:, 0, :] + g1 * rk_ref[:, 1, :]


def _combine(xo, rk3, gates):
    grid = (S // BS,)
    row = pl.BlockSpec((BS, H), lambda i: (i, 0))
    return pl.pallas_call(
        _combine_body,
        grid=grid,
        in_specs=[row,
                  pl.BlockSpec((BS, TOPK, H), lambda i: (i, 0, 0)),
                  pl.BlockSpec((BS, LANES), lambda i: (i, 0))],
        out_specs=row,
        out_shape=jax.ShapeDtypeStruct((S, H), jnp.float32),
        interpret=_INTERPRET,
    )(xo, rk3, gates)


_TOK_TILE = T // SC_TILES


def _dispatch_body(topi0_hbm, topi1_hbm, rank0_hbm, rank1_hbm, off_hbm,
                   h2_hbm, h2s_hbm, dest0_hbm, dest1_hbm,
                   topi0_v, topi1_v, rank0_v, rank1_v, off_v,
                   d0_v, d1_v, rows_v, sem):
    cid = lax.axis_index("c")
    sid = lax.axis_index("s")
    wid = sid * SC_CORES + cid
    tbase = wid * _TOK_TILE

    pltpu.sync_copy(topi0_hbm.at[pl.ds(tbase, _TOK_TILE)], topi0_v)
    pltpu.sync_copy(topi1_hbm.at[pl.ds(tbase, _TOK_TILE)], topi1_v)
    pltpu.sync_copy(rank0_hbm.at[pl.ds(tbase, _TOK_TILE)], rank0_v)
    pltpu.sync_copy(rank1_hbm.at[pl.ds(tbase, _TOK_TILE)], rank1_v)
    pltpu.sync_copy(off_hbm, off_v)

    def body(i, carry):
        e0 = topi0_v[pl.ds(i * 16, 16)]
        d0_v[pl.ds(i * 16, 16)] = (plsc.load_gather(off_v, [e0])
                                   + rank0_v[pl.ds(i * 16, 16)])
        e1 = topi1_v[pl.ds(i * 16, 16)]
        d1_v[pl.ds(i * 16, 16)] = (plsc.load_gather(off_v, [e1])
                                   + rank1_v[pl.ds(i * 16, 16)])
        return carry
    lax.fori_loop(0, _TOK_TILE // 16, body, 0)

    pltpu.sync_copy(d0_v, dest0_hbm.at[pl.ds(tbase, _TOK_TILE)])
    pltpu.sync_copy(d1_v, dest1_hbm.at[pl.ds(tbase, _TOK_TILE)])

    pltpu.sync_copy(h2_hbm.at[pl.ds(tbase, _TOK_TILE)], rows_v)
    pltpu.sync_copy(rows_v, h2s_hbm.at[d0_v])
    pltpu.sync_copy(rows_v, h2s_hbm.at[d1_v])


def _dispatch(topi0, topi1, rank0, rank1, off16, h2):
    mesh = plsc.VectorSubcoreMesh(core_axis_name="c", subcore_axis_name="s")
    fn = functools.partial(
        pl.kernel, _dispatch_body, mesh=mesh,
        compiler_params=pltpu.CompilerParams(needs_layout_passes=False, use_tc_tiling_on_sc=True),
        out_type=[jax.ShapeDtypeStruct((APAD, H), jnp.float32),
                  jax.ShapeDtypeStruct((T,), jnp.int32),
                  jax.ShapeDtypeStruct((T,), jnp.int32)],
        scratch_types=[
            pltpu.VMEM((_TOK_TILE,), jnp.int32),
            pltpu.VMEM((_TOK_TILE,), jnp.int32),
            pltpu.VMEM((_TOK_TILE,), jnp.int32),
            pltpu.VMEM((_TOK_TILE,), jnp.int32),
            pltpu.VMEM((16,), jnp.int32),
            pltpu.VMEM((_TOK_TILE,), jnp.int32),
            pltpu.VMEM((_TOK_TILE,), jnp.int32),
            pltpu.VMEM((_TOK_TILE, H), jnp.float32),
            pltpu.SemaphoreType.DMA,
        ],
    )()
    return fn(topi0, topi1, rank0, rank1, off16, h2)


_CPER_TILE = A // SC_TILES
_CCHUNK = _CPER_TILE // 2


def _cgather_body(dest_hbm, eo_hbm, rk_hbm, idx0_v, idx1_v, rows_v, sem):
    cid = lax.axis_index("c")
    sid = lax.axis_index("s")
    wid = sid * SC_CORES + cid
    base = wid * _CPER_TILE
    for ch, idx_v in enumerate((idx0_v, idx1_v)):
        pltpu.sync_copy(dest_hbm.at[pl.ds(base + ch * _CCHUNK, _CCHUNK)], idx_v)
        pltpu.async_copy(eo_hbm.at[idx_v], rows_v, sem).wait()
        pltpu.sync_copy(rows_v, rk_hbm.at[pl.ds(base + ch * _CCHUNK, _CCHUNK)])


def _cgather(dest, eo_s):
    mesh = plsc.VectorSubcoreMesh(core_axis_name="c", subcore_axis_name="s")
    fn = functools.partial(
        pl.kernel, _cgather_body, mesh=mesh,
        compiler_params=pltpu.CompilerParams(needs_layout_passes=False, use_tc_tiling_on_sc=True),
        out_type=jax.ShapeDtypeStruct((A, H), jnp.float32),
        scratch_types=[
            pltpu.VMEM((_CCHUNK,), jnp.int32),
            pltpu.VMEM((_CCHUNK,), jnp.int32),
            pltpu.VMEM((_CCHUNK, H), jnp.float32),
            pltpu.SemaphoreType.DMA,
        ],
    )()
    return fn(dest, eo_s)


def _rope_tables():
    pos = jnp.arange(S, dtype=jnp.float32)
    inv = 1.0 / (10000.0 ** (jnp.arange(0, DH, 2, dtype=jnp.float32) / DH))
    ang = pos[:, None] * inv[None, :]
    cos = jnp.repeat(jnp.cos(ang), 2, axis=1)
    sin = jnp.repeat(jnp.sin(ang), 2, axis=1)
    return jnp.tile(cos, (1, NH)), jnp.tile(sin, (1, NH))


def kernel(x, attn_norm_g, Wq, Wk, Wv, Wo, ffn_norm_g, router_W, W1, W2):
    x2d = x.reshape(T, H)
    g = attn_norm_g.reshape(1, H)
    g2 = ffn_norm_g.reshape(1, H)
    cos_t, sin_t = _rope_tables()

    q, k, v = _qkv(x2d, g, Wq, Wk, Wv, cos_t, sin_t)
    qh = q.reshape(S, NH, DH).transpose(1, 0, 2)
    kh = k.reshape(S, NH, DH).transpose(1, 0, 2)
    vh = v.reshape(S, NH, DH).transpose(1, 0, 2)
    oh = _attention(qh, kh, vh)
    o2d = oh.transpose(1, 0, 2).reshape(S, H)

    wr_pad = jnp.zeros((H, LANES), jnp.float32).at[:, :E].set(router_W)
    xo, h2, meta, gates, psum, cnts = _router(x2d, o2d, Wo, g2, wr_pad)

    counts = cnts[0, :E]

    aligned = ((counts.astype(jnp.int32) + BM - 1) // BM) * BM
    po = jnp.cumsum(aligned) - aligned
    off16 = jnp.zeros((16,), jnp.int32).at[:E].set(po)
    chunk_start = jnp.arange(NCHUNK, dtype=jnp.int32) * BM
    e_idx = jnp.sum(chunk_start[:, None] >= po[None, :], axis=1).astype(jnp.int32) - 1

    h2s, dest0, dest1 = _dispatch(meta[:, 0], meta[:, 1], meta[:, 2],
                                  meta[:, 3], off16, h2)
    eo_s = _grouped_ffn(h2s, W1, W2, e_idx)
    dest = jnp.stack([dest0, dest1], axis=1).reshape(A)
    rk = _cgather(dest, eo_s)
    out2d = _combine(xo, rk.reshape(S, TOPK, H), gates)

    pmean = psum[0, :E] / T
    frac = counts / T
    aux = (E * jnp.sum(frac * pmean)).astype(jnp.float32)
    return (out2d.reshape(B, S, H), aux)
